# Initial kernel scaffold; baseline (speedup 1.0000x reference)
import jax
import jax.numpy as jnp
from jax.experimental import pallas as pl


def kernel(x, gate_values, W1, b1, W2, b2):
    raise NotImplementedError("write your pallas kernel here")


# speedup vs baseline: 3.3766x; 3.3766x over previous
import functools

import jax
import jax.numpy as jnp
import numpy as np
from jax.experimental import pallas as pl
from jax.experimental.pallas import tpu as pltpu

C = 96
H = 224
W = 224
B = 4
HT = 28
T = H // HT
R1 = HT + 8
S0 = R1 * W
S1 = (HT + 2) * W
S2 = HT * W


def _body(x_hbm, x_res, geff_ref, wm1, wm2, b1c, b2c, aux,
          out_ref, x_scr, sem):
    b = pl.program_id(0)
    t = pl.program_id(1)

    @pl.when(jnp.logical_and(t > 0, t < T - 1))
    def _interior():
        cp = pltpu.make_async_copy(
            x_hbm.at[b, :, pl.ds((t * HT - 4) * W, S0)], x_scr, sem)
        cp.start()
        cp.wait()

    @pl.when(t == 0)
    def _first():
        x_scr[:, 0:4 * W] = jnp.zeros((C, 4 * W), jnp.float32)
        cp = pltpu.make_async_copy(
            x_hbm.at[b, :, pl.ds(0, S0 - 4 * W)],
            x_scr.at[:, pl.ds(4 * W, S0 - 4 * W)], sem)
        cp.start()
        cp.wait()

    @pl.when(t == T - 1)
    def _last():
        cp = pltpu.make_async_copy(
            x_hbm.at[b, :, pl.ds((H - HT - 4) * W, S0 - 4 * W)],
            x_scr.at[:, pl.ds(0, S0 - 4 * W)], sem)
        cp.start()
        cp.wait()
        x_scr[:, S0 - 4 * W:S0] = jnp.zeros((C, 4 * W), jnp.float32)

    geff = geff_ref[:, :]

    m_l = aux[0:1, :]
    m_r = aux[1:2, :]
    rowi = aux[2:3, :]

    def conv_as_matmul(src_bf, wm_ref, s_out, base=0):
        xcol = jnp.concatenate(
            [src_bf[:, (base + ky) * W: (base + ky) * W + s_out]
             for ky in range(3)], axis=0)
        y = jax.lax.dot_general(
            wm_ref[:, :], xcol, (((1,), (0,)), ((), ())),
            preferred_element_type=jnp.float32)
        y0, y1, y2 = y[0:C], y[C:2 * C], y[2 * C:3 * C]
        zc = jnp.zeros((C, 1), jnp.float32)
        rs = jnp.concatenate([zc, y0[:, :-1]], axis=1)
        ls = jnp.concatenate([y2[:, 1:], zc], axis=1)
        return y1 + m_l[:, :s_out] * rs + m_r[:, :s_out] * ls

    xb = x_scr[:, :].astype(jnp.bfloat16)
    conv1 = conv_as_matmul(xb, wm1, S1, base=2)
    h1 = jnp.maximum(geff * (conv1 + b1c[:, :]), 0.0)
    rglob = rowi[:, :S1] + (t * HT - 1).astype(jnp.float32)
    h1 = jnp.where(jnp.logical_and(rglob >= 0.0, rglob <= H - 1.0), h1, 0.0)

    conv2 = conv_as_matmul(h1.astype(jnp.bfloat16), wm2, S2)
    out_ref[:, :] = jnp.maximum(geff * (conv2 + b2c[:, :]), 0.0) + x_res[:, :]


@functools.partial(jax.jit, static_argnums=())
def kernel(x, gate_values, W1, b1, W2, b2):
    xr = x.reshape(B, C, H * W)
    geff = jnp.where(jnp.sum(gate_values, axis=0, keepdims=True) > 0,
                     jnp.maximum(gate_values, 0.0), 0.0)
    geff = geff.reshape(B, C, 1)
    wm1 = W1.transpose(3, 0, 2, 1).reshape(3 * C, 3 * C).astype(jnp.bfloat16)
    wm2 = W2.transpose(3, 0, 2, 1).reshape(3 * C, 3 * C).astype(jnp.bfloat16)
    b1c = b1.reshape(C, 1)
    b2c = b2.reshape(C, 1)
    w_idx = np.arange(S1) % W
    aux = jnp.asarray(np.stack([
        (w_idx >= 1).astype(np.float32),
        (w_idx <= W - 2).astype(np.float32),
        (np.arange(S1) // W).astype(np.float32),
    ]))

    grid = (B, T)
    out = pl.pallas_call(
        _body,
        grid=grid,
        in_specs=[
            pl.BlockSpec(memory_space=pl.ANY),
            pl.BlockSpec((None, C, S2), lambda b, t: (b, 0, t)),
            pl.BlockSpec((None, C, 1), lambda b, t: (b, 0, 0)),
            pl.BlockSpec((3 * C, 3 * C), lambda b, t: (0, 0)),
            pl.BlockSpec((3 * C, 3 * C), lambda b, t: (0, 0)),
            pl.BlockSpec((C, 1), lambda b, t: (0, 0)),
            pl.BlockSpec((C, 1), lambda b, t: (0, 0)),
            pl.BlockSpec((3, S1), lambda b, t: (0, 0)),
        ],
        out_specs=pl.BlockSpec((None, C, S2), lambda b, t: (b, 0, t)),
        out_shape=jax.ShapeDtypeStruct((B, C, H * W), jnp.float32),
        scratch_shapes=[
            pltpu.VMEM((C, S0), jnp.float32),
            pltpu.SemaphoreType.DMA,
        ],
    )(xr, xr, geff, wm1, wm2, b1c, b2c, aux)
    return out.reshape(B, C, H, W)


# speedup vs baseline: 3.4849x; 1.0321x over previous
import functools

import jax
import jax.numpy as jnp
import numpy as np
from jax.experimental import pallas as pl
from jax.experimental.pallas import tpu as pltpu

C = 96
H = 224
W = 224
B = 4
HT = 56
T = H // HT
R1 = HT + 8
S0 = R1 * W
S1 = (HT + 2) * W
S2 = HT * W


def _body(x_hbm, x_res, geff_ref, wm1, wm2, b1c, b2c, aux,
          out_ref, x_scr, sem):
    b = pl.program_id(0)
    t = pl.program_id(1)

    @pl.when(jnp.logical_and(t > 0, t < T - 1))
    def _interior():
        cp = pltpu.make_async_copy(
            x_hbm.at[b, :, pl.ds((t * HT - 4) * W, S0)], x_scr, sem)
        cp.start()
        cp.wait()

    @pl.when(t == 0)
    def _first():
        x_scr[:, 0:4 * W] = jnp.zeros((C, 4 * W), jnp.float32)
        cp = pltpu.make_async_copy(
            x_hbm.at[b, :, pl.ds(0, S0 - 4 * W)],
            x_scr.at[:, pl.ds(4 * W, S0 - 4 * W)], sem)
        cp.start()
        cp.wait()

    @pl.when(t == T - 1)
    def _last():
        cp = pltpu.make_async_copy(
            x_hbm.at[b, :, pl.ds((H - HT - 4) * W, S0 - 4 * W)],
            x_scr.at[:, pl.ds(0, S0 - 4 * W)], sem)
        cp.start()
        cp.wait()
        x_scr[:, S0 - 4 * W:S0] = jnp.zeros((C, 4 * W), jnp.float32)

    geff = geff_ref[:, :]

    m_l = aux[0:1, :]
    m_r = aux[1:2, :]
    rowi = aux[2:3, :]

    def conv_as_matmul(src_bf, wm_ref, s_out, base=0):
        xcol = jnp.concatenate(
            [src_bf[:, (base + ky) * W: (base + ky) * W + s_out]
             for ky in range(3)], axis=0)
        y = jax.lax.dot_general(
            wm_ref[:, :], xcol, (((1,), (0,)), ((), ())),
            preferred_element_type=jnp.float32)
        y0, y1, y2 = y[0:C], y[C:2 * C], y[2 * C:3 * C]
        zc = jnp.zeros((C, 1), jnp.float32)
        rs = jnp.concatenate([zc, y0[:, :-1]], axis=1)
        ls = jnp.concatenate([y2[:, 1:], zc], axis=1)
        return y1 + m_l[:, :s_out] * rs + m_r[:, :s_out] * ls

    xb = x_scr[:, :].astype(jnp.bfloat16)
    conv1 = conv_as_matmul(xb, wm1, S1, base=2)
    h1 = jnp.maximum(geff * (conv1 + b1c[:, :]), 0.0)
    rglob = rowi[:, :S1] + (t * HT - 1).astype(jnp.float32)
    h1 = jnp.where(jnp.logical_and(rglob >= 0.0, rglob <= H - 1.0), h1, 0.0)

    conv2 = conv_as_matmul(h1.astype(jnp.bfloat16), wm2, S2)
    out_ref[:, :] = jnp.maximum(geff * (conv2 + b2c[:, :]), 0.0) + x_res[:, :]


@functools.partial(jax.jit, static_argnums=())
def kernel(x, gate_values, W1, b1, W2, b2):
    xr = x.reshape(B, C, H * W)
    geff = jnp.where(jnp.sum(gate_values, axis=0, keepdims=True) > 0,
                     jnp.maximum(gate_values, 0.0), 0.0)
    geff = geff.reshape(B, C, 1)
    wm1 = W1.transpose(3, 0, 2, 1).reshape(3 * C, 3 * C).astype(jnp.bfloat16)
    wm2 = W2.transpose(3, 0, 2, 1).reshape(3 * C, 3 * C).astype(jnp.bfloat16)
    b1c = b1.reshape(C, 1)
    b2c = b2.reshape(C, 1)
    w_idx = np.arange(S1) % W
    aux = jnp.asarray(np.stack([
        (w_idx >= 1).astype(np.float32),
        (w_idx <= W - 2).astype(np.float32),
        (np.arange(S1) // W).astype(np.float32),
    ]))

    grid = (B, T)
    out = pl.pallas_call(
        _body,
        grid=grid,
        in_specs=[
            pl.BlockSpec(memory_space=pl.ANY),
            pl.BlockSpec((None, C, S2), lambda b, t: (b, 0, t)),
            pl.BlockSpec((None, C, 1), lambda b, t: (b, 0, 0)),
            pl.BlockSpec((3 * C, 3 * C), lambda b, t: (0, 0)),
            pl.BlockSpec((3 * C, 3 * C), lambda b, t: (0, 0)),
            pl.BlockSpec((C, 1), lambda b, t: (0, 0)),
            pl.BlockSpec((C, 1), lambda b, t: (0, 0)),
            pl.BlockSpec((3, S1), lambda b, t: (0, 0)),
        ],
        out_specs=pl.BlockSpec((None, C, S2), lambda b, t: (b, 0, t)),
        out_shape=jax.ShapeDtypeStruct((B, C, H * W), jnp.float32),
        scratch_shapes=[
            pltpu.VMEM((C, S0), jnp.float32),
            pltpu.SemaphoreType.DMA,
        ],
    )(xr, xr, geff, wm1, wm2, b1c, b2c, aux)
    return out.reshape(B, C, H, W)
